# baseline (speedup 1.0000x reference)
import numpy as np
import jax
import jax.numpy as jnp
from jax.experimental import pallas as pl

_NODES = 64
_PRED_STEPS = 8


def _perm_matrix(nodes, t_keep, dims):
    f = nodes * t_keep * dims
    p = np.zeros((f, f), np.float32)
    n, t, d = np.meshgrid(np.arange(nodes), np.arange(t_keep), np.arange(dims),
                          indexing="ij")
    rows = (n * t_keep * dims + t * dims + d).ravel()
    cols = (t * nodes * dims + n * dims + d).ravel()
    p[rows, cols] = 1.0
    return p


def _elu(x):
    return jnp.where(x > 0, x, jnp.exp(jnp.minimum(x, 0.0)) - 1.0)


def _predict_kernel(x_ref, p_ref, w1_ref, b1_ref, w2_ref, b2_ref, bnw_ref,
                    bnb_ref, wo_ref, bo_ref, o_ref):
    nb, tk, feat = o_ref.shape
    rows = nb * tk
    xp = jnp.dot(x_ref[...], p_ref[...], preferred_element_type=jnp.float32)
    x = xp.reshape(rows, feat)
    w1 = w1_ref[...].T
    b1 = b1_ref[...]
    w2 = w2_ref[...].T
    b2 = b2_ref[...]
    bnw = bnw_ref[...]
    bnb = bnb_ref[...]
    wo = wo_ref[...].T
    bo = bo_ref[...]

    def step(_, x):
        h = jnp.dot(x, w1, preferred_element_type=jnp.float32) + b1
        h = _elu(h)
        h = jnp.dot(h, w2, preferred_element_type=jnp.float32) + b2
        h = _elu(h)
        mean = jnp.sum(h, axis=0, keepdims=True) * (1.0 / rows)
        c = h - mean
        var = jnp.sum(c * c, axis=0, keepdims=True) * (1.0 / rows)
        h = c * jax.lax.rsqrt(var + 1e-5) * bnw + bnb
        out = jnp.dot(h, wo, preferred_element_type=jnp.float32) + bo
        return x + out

    xf = jax.lax.fori_loop(0, _PRED_STEPS, step, x, unroll=True)
    o_ref[...] = xf.reshape(o_ref.shape)


def kernel(inputs, edge_index, edges, fc1_w, fc1_b, fc2_w, fc2_b, bn_w, bn_b,
           fco_w, fco_b, prediction_steps):
    del edge_index, edges, prediction_steps
    nodes = _NODES
    dims = inputs.shape[-1]
    batch = inputs.shape[0] // nodes
    timesteps = inputs.shape[1]
    t_keep = (timesteps + _PRED_STEPS - 1) // _PRED_STEPS
    rows = batch * t_keep
    feat = nodes * dims
    xs = (jnp.take(inputs, jnp.arange(0, timesteps, _PRED_STEPS), axis=1)
          .reshape(batch, nodes * t_keep * dims))
    perm = jnp.asarray(_perm_matrix(nodes, t_keep, dims))

    out2d = pl.pallas_call(
        _predict_kernel,
        out_shape=jax.ShapeDtypeStruct((batch, t_keep, feat), jnp.float32),
    )(
        xs, perm,
        fc1_w, fc1_b.reshape(1, -1),
        fc2_w, fc2_b.reshape(1, -1),
        bn_w.reshape(1, -1), bn_b.reshape(1, -1),
        fco_w, fco_b.reshape(1, -1),
    )
    return out2d.reshape(batch, t_keep, nodes, dims)


# speedup vs baseline: 3.1217x; 3.1217x over previous
import numpy as np
import jax
import jax.numpy as jnp
from jax.experimental import pallas as pl

_NODES = 64
_PRED_STEPS = 8


def _perm_matrix(nodes, t_keep, dims):
    f = nodes * t_keep * dims
    p = np.zeros((f, f), np.float32)
    n, t, d = np.meshgrid(np.arange(nodes), np.arange(t_keep), np.arange(dims),
                          indexing="ij")
    rows = (n * t_keep * dims + t * dims + d).ravel()
    cols = (t * nodes * dims + n * dims + d).ravel()
    p[rows, cols] = 1.0
    return p


def _elu(x):
    return jnp.where(x > 0, x, jnp.exp(jnp.minimum(x, 0.0)) - 1.0)


def _predict_kernel(x_ref, p_ref, w1_ref, b1_ref, w2_ref, b2_ref, bnw_ref,
                    bnb_ref, wo_ref, bo_ref, o_ref):
    nb, tk, feat = o_ref.shape
    rows = nb * tk
    xp = jnp.dot(x_ref[...], p_ref[...], preferred_element_type=jnp.float32)
    x = xp.reshape(rows, feat)
    w1 = w1_ref[...].T
    b1 = b1_ref[...]
    w2 = w2_ref[...].T
    b2 = b2_ref[...]
    bnw = bnw_ref[...]
    bnb = bnb_ref[...]
    wo = wo_ref[...].T
    bo = bo_ref[...]

    def step(_, x):
        h = jnp.dot(x, w1, preferred_element_type=jnp.float32) + b1
        h = _elu(h)
        h = jnp.dot(h, w2, preferred_element_type=jnp.float32) + b2
        h = _elu(h)
        s1 = jnp.sum(h, axis=0, keepdims=True)
        s2 = jnp.sum(h * h, axis=0, keepdims=True)
        mean = s1 * (1.0 / rows)
        var = s2 * (1.0 / rows) - mean * mean
        g = jax.lax.rsqrt(var + 1e-5) * bnw
        wo_s = wo * g.reshape(-1, 1)
        bias = jnp.dot(bnb - mean * g, wo,
                       preferred_element_type=jnp.float32) + bo
        out = jnp.dot(h, wo_s, preferred_element_type=jnp.float32) + bias
        return x + out

    xf = jax.lax.fori_loop(0, _PRED_STEPS, step, x, unroll=True)
    o_ref[...] = xf.reshape(o_ref.shape)


def kernel(inputs, edge_index, edges, fc1_w, fc1_b, fc2_w, fc2_b, bn_w, bn_b,
           fco_w, fco_b, prediction_steps):
    del edge_index, edges, prediction_steps
    nodes = _NODES
    dims = inputs.shape[-1]
    batch = inputs.shape[0] // nodes
    timesteps = inputs.shape[1]
    t_keep = (timesteps + _PRED_STEPS - 1) // _PRED_STEPS
    rows = batch * t_keep
    feat = nodes * dims
    xs = (inputs.reshape(batch, nodes, timesteps, dims)[:, :, ::_PRED_STEPS, :]
          .reshape(batch, nodes * t_keep * dims))
    perm = jnp.asarray(_perm_matrix(nodes, t_keep, dims))

    out2d = pl.pallas_call(
        _predict_kernel,
        out_shape=jax.ShapeDtypeStruct((batch, t_keep, feat), jnp.float32),
    )(
        xs, perm,
        fc1_w, fc1_b.reshape(1, -1),
        fc2_w, fc2_b.reshape(1, -1),
        bn_w.reshape(1, -1), bn_b.reshape(1, -1),
        fco_w, fco_b.reshape(1, -1),
    )
    return out2d.reshape(batch, t_keep, nodes, dims)


# speedup vs baseline: 3.1476x; 1.0083x over previous
import numpy as np
import jax
import jax.numpy as jnp
from jax.experimental import pallas as pl

_NODES = 64
_PRED_STEPS = 8


def _perm_matrix(nodes, t_keep, dims):
    f = nodes * t_keep * dims
    p = np.zeros((f, f), np.float32)
    n, t, d = np.meshgrid(np.arange(nodes), np.arange(t_keep), np.arange(dims),
                          indexing="ij")
    rows = (n * t_keep * dims + t * dims + d).ravel()
    cols = (t * nodes * dims + n * dims + d).ravel()
    p[rows, cols] = 1.0
    return p


def _elu(x):
    return jnp.where(x > 0, x, jnp.exp(x) - 1.0)


def _predict_kernel(x_ref, p_ref, w1_ref, b1_ref, w2_ref, b2_ref, bnw_ref,
                    bnb_ref, wo_ref, bo_ref, o_ref):
    nb, tk, feat = o_ref.shape
    rows = nb * tk
    xp = jnp.dot(x_ref[...], p_ref[...], preferred_element_type=jnp.float32)
    x = xp.reshape(rows, feat)
    w1 = w1_ref[...].T
    b1 = b1_ref[...]
    w2 = w2_ref[...].T
    b2 = b2_ref[...]
    bnw = bnw_ref[...]
    bnb = bnb_ref[...]
    wo = wo_ref[...].T
    bo = bo_ref[...]

    def step(_, x):
        h = jnp.dot(x, w1, preferred_element_type=jnp.float32) + b1
        h = _elu(h)
        h = jnp.dot(h, w2, preferred_element_type=jnp.float32) + b2
        h = _elu(h)
        s1 = jnp.sum(h, axis=0, keepdims=True)
        s2 = jnp.sum(h * h, axis=0, keepdims=True)
        mean = s1 * (1.0 / rows)
        var = s2 * (1.0 / rows) - mean * mean
        g = jax.lax.rsqrt(var + 1e-5) * bnw
        wo_s = wo * g.reshape(-1, 1)
        bias = jnp.dot(bnb - mean * g, wo,
                       preferred_element_type=jnp.float32) + bo
        out = jnp.dot(h, wo_s, preferred_element_type=jnp.float32) + bias
        return x + out

    xf = jax.lax.fori_loop(0, _PRED_STEPS, step, x, unroll=True)
    o_ref[...] = xf.reshape(o_ref.shape)


def kernel(inputs, edge_index, edges, fc1_w, fc1_b, fc2_w, fc2_b, bn_w, bn_b,
           fco_w, fco_b, prediction_steps):
    del edge_index, edges, prediction_steps
    nodes = _NODES
    dims = inputs.shape[-1]
    batch = inputs.shape[0] // nodes
    timesteps = inputs.shape[1]
    t_keep = (timesteps + _PRED_STEPS - 1) // _PRED_STEPS
    rows = batch * t_keep
    feat = nodes * dims
    xs = (inputs.reshape(batch, nodes, timesteps, dims)[:, :, ::_PRED_STEPS, :]
          .reshape(batch, nodes * t_keep * dims))
    perm = jnp.asarray(_perm_matrix(nodes, t_keep, dims))

    out2d = pl.pallas_call(
        _predict_kernel,
        out_shape=jax.ShapeDtypeStruct((batch, t_keep, feat), jnp.float32),
    )(
        xs, perm,
        fc1_w, fc1_b.reshape(1, -1),
        fc2_w, fc2_b.reshape(1, -1),
        bn_w.reshape(1, -1), bn_b.reshape(1, -1),
        fco_w, fco_b.reshape(1, -1),
    )
    return out2d.reshape(batch, t_keep, nodes, dims)
